# Initial kernel scaffold; baseline (speedup 1.0000x reference)
import jax
import jax.numpy as jnp
from jax.experimental import pallas as pl


def kernel(x, edge_index, W, b):
    raise NotImplementedError("write your pallas kernel here")


# speedup vs baseline: 30.2485x; 30.2485x over previous
import jax
import jax.numpy as jnp
from jax import lax
from jax.experimental import pallas as pl
from jax.experimental.pallas import tpu as pltpu
from jax.experimental.pallas import tpu_sc as plsc

N = 10000
D = 128
NP = 10240
NPT = NP // 16
EC = 128
RPT = 80
NTILES = 32
EROWS = NTILES * RPT
EP = EROWS * EC
NC, NS = 2, 16

_mesh = plsc.VectorSubcoreMesh(core_axis_name="c", subcore_axis_name="s")


def _deg_body(dst_hbm, out_hbm, idx_v, ones_v, zero_v, hist_sm):
    c = lax.axis_index("c")
    s = lax.axis_index("s")
    wid = s * NC + c
    for i in range(EC // 16):
        ones_v[pl.ds(i * 16, 16)] = jnp.full((16,), 1.0, jnp.float32)
    for i in range(NPT // 16):
        zero_v[pl.ds(i * 16, 16)] = jnp.zeros((16,), jnp.float32)
    pltpu.sync_copy(zero_v, hist_sm.at[pl.ds(s * NPT, NPT)])
    plsc.subcore_barrier()
    pltpu.sync_copy(dst_hbm.at[pl.ds(wid * RPT, RPT)], idx_v)

    def body(j, carry):
        pltpu.sync_copy(ones_v, hist_sm.at[idx_v.at[j]], add=True)
        return carry

    lax.fori_loop(0, RPT, body, 0)
    plsc.subcore_barrier()
    pltpu.sync_copy(hist_sm.at[pl.ds(s * NPT, NPT)],
                    out_hbm.at[c, pl.ds(s * NPT, NPT)])


_deg_call = pl.kernel(
    _deg_body,
    out_type=jax.ShapeDtypeStruct((NC, NP), jnp.float32),
    mesh=_mesh,
    scratch_types=[
        pltpu.VMEM((RPT, EC), jnp.int32),
        pltpu.VMEM((EC,), jnp.float32),
        pltpu.VMEM((NPT,), jnp.float32),
        pltpu.VMEM_SHARED((NP,), jnp.float32),
    ],
)


def _scat_body(g_hbm, src_hbm, dst_hbm, out_hbm, src_v, dst_v, rows_v, acc_sm):
    c = lax.axis_index("c")
    s = lax.axis_index("s")
    wid = s * NC + c
    pltpu.sync_copy(g_hbm.at[pl.ds(s * NPT, NPT)], acc_sm.at[pl.ds(s * NPT, NPT)])
    plsc.subcore_barrier()
    pltpu.sync_copy(src_hbm.at[pl.ds(wid * RPT, RPT)], src_v)
    pltpu.sync_copy(dst_hbm.at[pl.ds(wid * RPT, RPT)], dst_v)

    def body(j, carry):
        pltpu.sync_copy(g_hbm.at[src_v.at[j]], rows_v)
        pltpu.sync_copy(rows_v, acc_sm.at[dst_v.at[j]], add=True)
        return carry

    lax.fori_loop(0, RPT, body, 0)
    plsc.subcore_barrier()
    pltpu.sync_copy(acc_sm.at[pl.ds(s * NPT, NPT)],
                    out_hbm.at[c, pl.ds(s * NPT, NPT)])


_scat_call = pl.kernel(
    _scat_body,
    out_type=jax.ShapeDtypeStruct((NC, NP, D), jnp.float32),
    mesh=_mesh,
    scratch_types=[
        pltpu.VMEM((RPT, EC), jnp.int32),
        pltpu.VMEM((RPT, EC), jnp.int32),
        pltpu.VMEM((EC, D), jnp.float32),
        pltpu.VMEM_SHARED((NP, D), jnp.float32),
    ],
)


def _mm_body(x_ref, w_ref, h0_ref, h1_ref, g_ref, dinv_ref):
    deg = 1.0 + h0_ref[...] + h1_ref[...]
    dinv = lax.rsqrt(deg)
    h = jnp.dot(x_ref[...], w_ref[...], preferred_element_type=jnp.float32)
    g_ref[...] = h * dinv
    dinv_ref[...] = dinv


_mm_call = pl.pallas_call(
    _mm_body,
    out_shape=[
        jax.ShapeDtypeStruct((NP, D), jnp.float32),
        jax.ShapeDtypeStruct((NP, 1), jnp.float32),
    ],
)


def _fin_body(p0_ref, p1_ref, g_ref, dinv_ref, b_ref, o_ref):
    ssum = p0_ref[...] + p1_ref[...] - g_ref[...]
    o = ssum * dinv_ref[...] + b_ref[...]
    o_ref[...] = jnp.maximum(o, 0.0)


_fin_call = pl.pallas_call(
    _fin_body,
    out_shape=jax.ShapeDtypeStruct((NP, D), jnp.float32),
)


def kernel(x, edge_index, W, b):
    src = edge_index[0]
    dst = edge_index[1]
    pad = EP - src.shape[0]
    pad_idx = (N + (jnp.arange(pad, dtype=jnp.int32) % (NP - N))).astype(jnp.int32)
    src_p = jnp.concatenate([src, pad_idx]).reshape(EROWS, EC)
    dst_p = jnp.concatenate([dst, pad_idx]).reshape(EROWS, EC)
    x_p = jnp.zeros((NP, D), x.dtype).at[:N].set(x)

    hist = _deg_call(dst_p)
    h0 = hist[0].reshape(NP, 1)
    h1 = hist[1].reshape(NP, 1)
    g, dinv = _mm_call(x_p, W, h0, h1)
    parts = _scat_call(g, src_p, dst_p)
    out = _fin_call(parts[0], parts[1], g, dinv, b.reshape(1, D))
    return out[:N]


# speedup vs baseline: 37.4628x; 1.2385x over previous
import jax
import jax.numpy as jnp
from jax import lax
from jax.experimental import pallas as pl
from jax.experimental.pallas import tpu as pltpu
from jax.experimental.pallas import tpu_sc as plsc

N = 10000
D = 128
NP = 10240
NPT = NP // 16
EC = 128
RPT = 80
HR = RPT // 2
NTILES = 32
EROWS = NTILES * RPT
EP = EROWS * EC
NC, NS = 2, 16

_mesh = plsc.VectorSubcoreMesh(core_axis_name="c", subcore_axis_name="s")


def _deg_body(dst_hbm, out_hbm, idx_v, ones_v, zero_v, hist_sm, sem):
    c = lax.axis_index("c")
    s = lax.axis_index("s")
    wid = s * NC + c
    for i in range(EC // 16):
        ones_v[pl.ds(i * 16, 16)] = jnp.full((16,), 1.0, jnp.float32)
    for i in range(NPT // 16):
        zero_v[pl.ds(i * 16, 16)] = jnp.zeros((16,), jnp.float32)
    pltpu.sync_copy(zero_v, hist_sm.at[pl.ds(s * NPT, NPT)])
    plsc.subcore_barrier()
    pltpu.sync_copy(dst_hbm.at[pl.ds(wid * RPT, RPT)], idx_v)

    def body(j, carry):
        pltpu.async_copy(ones_v, hist_sm.at[idx_v.at[j]], sem, add=True)
        return carry

    lax.fori_loop(0, RPT, body, 0)
    pltpu.make_async_copy(dst_hbm.at[pl.ds(0, RPT)], idx_v, sem).wait()
    plsc.subcore_barrier()
    pltpu.sync_copy(hist_sm.at[pl.ds(s * NPT, NPT)],
                    out_hbm.at[c, pl.ds(s * NPT, NPT)])


_deg_call = pl.kernel(
    _deg_body,
    out_type=jax.ShapeDtypeStruct((NC, NP), jnp.float32),
    mesh=_mesh,
    scratch_types=[
        pltpu.VMEM((RPT, EC), jnp.int32),
        pltpu.VMEM((EC,), jnp.float32),
        pltpu.VMEM((NPT,), jnp.float32),
        pltpu.VMEM_SHARED((NP,), jnp.float32),
        pltpu.SemaphoreType.DMA,
    ],
)


def _scat_body(g_hbm, src_hbm, dst_hbm, out_hbm, src_v, dst_v, buf_p, buf_q,
               acc_sm, sem_i, sem_x, sem_gp, sem_gq, sem_sp, sem_sq):
    c = lax.axis_index("c")
    s = lax.axis_index("s")
    wid = s * NC + c
    init_d = pltpu.async_copy(g_hbm.at[pl.ds(s * NPT, NPT)],
                              acc_sm.at[pl.ds(s * NPT, NPT)], sem_i)

    def g_fire(j, buf, sem):
        pltpu.async_copy(g_hbm.at[src_v.at[j]], buf, sem)

    def g_drain(buf, sem):
        pltpu.make_async_copy(g_hbm.at[pl.ds(0, EC)], buf, sem).wait()

    def s_fire(j, buf, sem):
        pltpu.async_copy(buf, acc_sm.at[dst_v.at[j]], sem, add=True)

    def s_drain(buf, sem):
        pltpu.make_async_copy(g_hbm.at[pl.ds(0, EC)], buf, sem).wait()

    def body(i, carry):
        jp = 2 * i
        jq = 2 * i + 1
        g_drain(buf_p, sem_gp)

        @pl.when(i > 0)
        def _():
            s_drain(buf_q, sem_sq)

        g_fire(jq, buf_q, sem_gq)
        s_fire(jp, buf_p, sem_sp)
        g_drain(buf_q, sem_gq)
        s_drain(buf_p, sem_sp)

        @pl.when(jq + 1 < HR)
        def _():
            g_fire(jq + 1, buf_p, sem_gp)

        s_fire(jq, buf_q, sem_sq)
        return carry

    for half in range(2):
        d1 = pltpu.async_copy(src_hbm.at[pl.ds(wid * RPT + half * HR, HR)],
                              src_v, sem_x)
        d2 = pltpu.async_copy(dst_hbm.at[pl.ds(wid * RPT + half * HR, HR)],
                              dst_v, sem_x)
        d1.wait()
        d2.wait()
        if half == 0:
            init_d.wait()
            plsc.subcore_barrier()
        g_fire(0, buf_p, sem_gp)
        lax.fori_loop(0, HR // 2, body, 0)
        s_drain(buf_q, sem_sq)

    plsc.subcore_barrier()
    pltpu.sync_copy(acc_sm.at[pl.ds(s * NPT, NPT)],
                    out_hbm.at[c, pl.ds(s * NPT, NPT)])


_scat_call = pl.kernel(
    _scat_body,
    out_type=jax.ShapeDtypeStruct((NC, NP, D), jnp.float32),
    mesh=_mesh,
    scratch_types=[
        pltpu.VMEM((HR, EC), jnp.int32),
        pltpu.VMEM((HR, EC), jnp.int32),
        pltpu.VMEM((EC, D), jnp.float32),
        pltpu.VMEM((EC, D), jnp.float32),
        pltpu.VMEM_SHARED((NP, D), jnp.float32),
        pltpu.SemaphoreType.DMA,
        pltpu.SemaphoreType.DMA,
        pltpu.SemaphoreType.DMA,
        pltpu.SemaphoreType.DMA,
        pltpu.SemaphoreType.DMA,
        pltpu.SemaphoreType.DMA,
    ],
)


def _mm_body(x_ref, w_ref, h_ref):
    h_ref[...] = jnp.dot(x_ref[...], w_ref[...],
                         preferred_element_type=jnp.float32)


_mm_call = pl.pallas_call(
    _mm_body,
    out_shape=jax.ShapeDtypeStruct((NP, D), jnp.float32),
)


def _scale_body(h_ref, h0_ref, h1_ref, g_ref, dinv_ref):
    deg = 1.0 + h0_ref[...] + h1_ref[...]
    dinv = lax.rsqrt(deg)
    g_ref[...] = h_ref[...] * dinv
    dinv_ref[...] = dinv


_scale_call = pl.pallas_call(
    _scale_body,
    out_shape=[
        jax.ShapeDtypeStruct((NP, D), jnp.float32),
        jax.ShapeDtypeStruct((NP, 1), jnp.float32),
    ],
)


def _fin_body(p0_ref, p1_ref, g_ref, dinv_ref, b_ref, o_ref):
    ssum = p0_ref[...] + p1_ref[...] - g_ref[...]
    o = ssum * dinv_ref[...] + b_ref[...]
    o_ref[...] = jnp.maximum(o, 0.0)


_fin_call = pl.pallas_call(
    _fin_body,
    out_shape=jax.ShapeDtypeStruct((NP, D), jnp.float32),
)


def kernel(x, edge_index, W, b):
    src = edge_index[0]
    dst = edge_index[1]
    pad = EP - src.shape[0]
    pad_idx = (N + (jnp.arange(pad, dtype=jnp.int32) % (NP - N))).astype(jnp.int32)
    src_p = jnp.concatenate([src, pad_idx]).reshape(EROWS, EC)
    dst_p = jnp.concatenate([dst, pad_idx]).reshape(EROWS, EC)
    x_p = jnp.zeros((NP, D), x.dtype).at[:N].set(x)

    hist = _deg_call(dst_p)
    h = _mm_call(x_p, W)
    h0 = hist[0].reshape(NP, 1)
    h1 = hist[1].reshape(NP, 1)
    g, dinv = _scale_call(h, h0, h1)
    parts = _scat_call(g, src_p, dst_p)
    out = _fin_call(parts[0], parts[1], g, dinv, b.reshape(1, D))
    return out[:N]


# speedup vs baseline: 38.7913x; 1.0355x over previous
import jax
import jax.numpy as jnp
from jax import lax
from jax.experimental import pallas as pl
from jax.experimental.pallas import tpu as pltpu
from jax.experimental.pallas import tpu_sc as plsc

N = 10000
D = 128
NP = 10240
NPT = NP // 16
EC = 128
RPT = 80
HR = RPT // 2
NTILES = 32
EROWS = NTILES * RPT
EP = EROWS * EC
NC, NS = 2, 16

_mesh = plsc.VectorSubcoreMesh(core_axis_name="c", subcore_axis_name="s")


def _deg_body(dst_hbm, out_hbm, idx_v, ones_v, zero_v, hist_sm, sem):
    c = lax.axis_index("c")
    s = lax.axis_index("s")
    wid = s * NC + c
    for i in range(EC // 16):
        ones_v[pl.ds(i * 16, 16)] = jnp.full((16,), 1.0, jnp.float32)
    for i in range(NPT // 16):
        zero_v[pl.ds(i * 16, 16)] = jnp.zeros((16,), jnp.float32)
    pltpu.sync_copy(zero_v, hist_sm.at[pl.ds(s * NPT, NPT)])
    plsc.subcore_barrier()
    pltpu.sync_copy(dst_hbm.at[pl.ds(wid * RPT, RPT)], idx_v)

    def body(j, carry):
        pltpu.async_copy(ones_v, hist_sm.at[idx_v.at[j]], sem, add=True)
        return carry

    lax.fori_loop(0, RPT, body, 0)
    pltpu.make_async_copy(dst_hbm.at[pl.ds(0, RPT)], idx_v, sem).wait()
    plsc.subcore_barrier()
    pltpu.sync_copy(hist_sm.at[pl.ds(s * NPT, NPT)],
                    out_hbm.at[c, pl.ds(s * NPT, NPT)])


_deg_call = pl.kernel(
    _deg_body,
    out_type=jax.ShapeDtypeStruct((NC, NP), jnp.float32),
    mesh=_mesh,
    scratch_types=[
        pltpu.VMEM((RPT, EC), jnp.int32),
        pltpu.VMEM((EC,), jnp.float32),
        pltpu.VMEM((NPT,), jnp.float32),
        pltpu.VMEM_SHARED((NP,), jnp.float32),
        pltpu.SemaphoreType.DMA,
    ],
)


def _scat_body(g_hbm, src_hbm, dst_hbm, out_hbm, src_v, dst_v, buf_p, buf_q,
               acc_sm, sem_i, sem_x, sem_gp, sem_gq, sem_sp, sem_sq):
    c = lax.axis_index("c")
    s = lax.axis_index("s")
    wid = s * NC + c
    init_d = pltpu.async_copy(g_hbm.at[pl.ds(s * NPT, NPT)],
                              acc_sm.at[pl.ds(s * NPT, NPT)], sem_i)

    def g_fire(j, buf, sem):
        pltpu.async_copy(g_hbm.at[src_v.at[j]], buf, sem)

    def g_drain(buf, sem):
        pltpu.make_async_copy(g_hbm.at[pl.ds(0, EC)], buf, sem).wait()

    def s_fire(j, buf, sem):
        pltpu.async_copy(buf, acc_sm.at[dst_v.at[j]], sem, add=True)

    def s_drain(buf, sem):
        pltpu.make_async_copy(g_hbm.at[pl.ds(0, EC)], buf, sem).wait()

    def body(i, carry):
        j0 = 2 * i
        j1 = 2 * i + 1

        @pl.when(i > 0)
        def _():
            s_drain(buf_q, sem_sq)
        g_fire(j1, buf_q, sem_gq)
        g_drain(buf_p, sem_gp)
        s_fire(j0, buf_p, sem_sp)
        g_drain(buf_q, sem_gq)
        s_drain(buf_p, sem_sp)

        @pl.when(j0 + 2 < HR)
        def _():
            g_fire(j0 + 2, buf_p, sem_gp)
        s_fire(j1, buf_q, sem_sq)
        return carry

    for half in range(2):
        d1 = pltpu.async_copy(src_hbm.at[pl.ds(wid * RPT + half * HR, HR)],
                              src_v, sem_x)
        d2 = pltpu.async_copy(dst_hbm.at[pl.ds(wid * RPT + half * HR, HR)],
                              dst_v, sem_x)
        d1.wait()
        d2.wait()
        if half == 0:
            init_d.wait()
            plsc.subcore_barrier()
        g_fire(0, buf_p, sem_gp)
        lax.fori_loop(0, HR // 2, body, 0)
        s_drain(buf_q, sem_sq)

    plsc.subcore_barrier()
    pltpu.sync_copy(acc_sm.at[pl.ds(s * NPT, NPT)],
                    out_hbm.at[c, pl.ds(s * NPT, NPT)])


_scat_call = pl.kernel(
    _scat_body,
    out_type=jax.ShapeDtypeStruct((NC, NP, D), jnp.float32),
    mesh=_mesh,
    scratch_types=[
        pltpu.VMEM((HR, EC), jnp.int32),
        pltpu.VMEM((HR, EC), jnp.int32),
        pltpu.VMEM((EC, D), jnp.float32),
        pltpu.VMEM((EC, D), jnp.float32),
        pltpu.VMEM_SHARED((NP, D), jnp.float32),
        pltpu.SemaphoreType.DMA,
        pltpu.SemaphoreType.DMA,
        pltpu.SemaphoreType.DMA,
        pltpu.SemaphoreType.DMA,
        pltpu.SemaphoreType.DMA,
        pltpu.SemaphoreType.DMA,
    ],
)


def _mm_body(x_ref, w_ref, h0_ref, h1_ref, g_ref, dinv_ref):
    deg = 1.0 + h0_ref[...] + h1_ref[...]
    dinv = lax.rsqrt(deg)
    h = jnp.dot(x_ref[...], w_ref[...], preferred_element_type=jnp.float32)
    g_ref[pl.ds(0, N), :] = h * dinv[:N, :]
    g_ref[pl.ds(N, NP - N), :] = jnp.zeros((NP - N, D), jnp.float32)
    dinv_ref[...] = dinv


_mm_call = pl.pallas_call(
    _mm_body,
    out_shape=[
        jax.ShapeDtypeStruct((NP, D), jnp.float32),
        jax.ShapeDtypeStruct((NP, 1), jnp.float32),
    ],
)


def _fin_body(p0_ref, p1_ref, g_ref, dinv_ref, b_ref, o_ref):
    ssum = p0_ref[...] + p1_ref[...] - g_ref[...]
    o = ssum * dinv_ref[...] + b_ref[...]
    o_ref[...] = jnp.maximum(o, 0.0)


_fin_call = pl.pallas_call(
    _fin_body,
    out_shape=jax.ShapeDtypeStruct((N, D), jnp.float32),
)


def kernel(x, edge_index, W, b):
    src = edge_index[0]
    dst = edge_index[1]
    pad = EP - src.shape[0]
    pad_idx = (N + (jnp.arange(pad, dtype=jnp.int32) % (NP - N))).astype(jnp.int32)
    src_p = jnp.concatenate([src, pad_idx]).reshape(EROWS, EC)
    dst_p = jnp.concatenate([dst, pad_idx]).reshape(EROWS, EC)

    hist = _deg_call(dst_p)
    h0 = hist[0].reshape(NP, 1)
    h1 = hist[1].reshape(NP, 1)
    g, dinv = _mm_call(x, W, h0, h1)
    parts = _scat_call(g, src_p, dst_p)
    out = _fin_call(parts[0, :N], parts[1, :N], g[:N], dinv[:N],
                    b.reshape(1, D))
    return out


# speedup vs baseline: 42.1725x; 1.0872x over previous
import jax
import jax.numpy as jnp
from jax import lax
from jax.experimental import pallas as pl
from jax.experimental.pallas import tpu as pltpu
from jax.experimental.pallas import tpu_sc as plsc

N = 10000
D = 128
NP = 10240
NPT = NP // 16
EC = 128
RPT = 80
HR = RPT // 2
NTILES = 32
EROWS = NTILES * RPT
EP = EROWS * EC
NC, NS = 2, 16

_mesh = plsc.VectorSubcoreMesh(core_axis_name="c", subcore_axis_name="s")


def _deg_body(dst_hbm, out_hbm, idx_v, ones_v, zero_v, hist_sm, sem):
    c = lax.axis_index("c")
    s = lax.axis_index("s")
    wid = s * NC + c
    for i in range(EC // 16):
        ones_v[pl.ds(i * 16, 16)] = jnp.full((16,), 1.0, jnp.float32)
    for i in range(NPT // 16):
        zero_v[pl.ds(i * 16, 16)] = jnp.zeros((16,), jnp.float32)
    pltpu.sync_copy(zero_v, hist_sm.at[pl.ds(s * NPT, NPT)])
    plsc.subcore_barrier()
    pltpu.sync_copy(dst_hbm.at[pl.ds(wid * RPT, RPT)], idx_v)

    def body(j, carry):
        pltpu.async_copy(ones_v, hist_sm.at[idx_v.at[j]], sem, add=True)
        return carry

    lax.fori_loop(0, RPT, body, 0)
    pltpu.make_async_copy(dst_hbm.at[pl.ds(0, RPT)], idx_v, sem).wait()
    plsc.subcore_barrier()
    pltpu.sync_copy(hist_sm.at[pl.ds(s * NPT, NPT)],
                    out_hbm.at[c, pl.ds(s * NPT, NPT)])


_deg_call = pl.kernel(
    _deg_body,
    out_type=jax.ShapeDtypeStruct((NC, NP), jnp.float32),
    mesh=_mesh,
    scratch_types=[
        pltpu.VMEM((RPT, EC), jnp.int32),
        pltpu.VMEM((EC,), jnp.float32),
        pltpu.VMEM((NPT,), jnp.float32),
        pltpu.VMEM_SHARED((NP,), jnp.float32),
        pltpu.SemaphoreType.DMA,
    ],
)


def _scat_body(g_hbm, src_hbm, dst_hbm, out_hbm, src_v, dst_v, buf_p, buf_q,
               acc_sm, sem_i, sem_x, sem_gp, sem_gq, sem_sp, sem_sq):
    c = lax.axis_index("c")
    s = lax.axis_index("s")
    wid = s * NC + c
    init_d = pltpu.async_copy(g_hbm.at[pl.ds(s * NPT, NPT)],
                              acc_sm.at[pl.ds(s * NPT, NPT)], sem_i)

    def g_fire(j, buf, sem):
        pltpu.async_copy(g_hbm.at[src_v.at[j]], buf, sem)

    def g_drain(buf, sem):
        pltpu.make_async_copy(g_hbm.at[pl.ds(0, EC)], buf, sem).wait()

    def s_fire(j, buf, sem):
        pltpu.async_copy(buf, acc_sm.at[dst_v.at[j]], sem, add=True)

    def s_drain(buf, sem):
        pltpu.make_async_copy(g_hbm.at[pl.ds(0, EC)], buf, sem).wait()

    def body(i, carry):
        j0 = 2 * i
        j1 = 2 * i + 1

        @pl.when(i > 0)
        def _():
            s_drain(buf_q, sem_sq)
        g_fire(j1, buf_q, sem_gq)
        g_drain(buf_p, sem_gp)
        s_fire(j0, buf_p, sem_sp)
        g_drain(buf_q, sem_gq)
        s_drain(buf_p, sem_sp)

        @pl.when(j0 + 2 < HR)
        def _():
            g_fire(j0 + 2, buf_p, sem_gp)
        s_fire(j1, buf_q, sem_sq)
        return carry

    for half in range(2):
        d1 = pltpu.async_copy(src_hbm.at[pl.ds(wid * RPT + half * HR, HR)],
                              src_v, sem_x)
        d2 = pltpu.async_copy(dst_hbm.at[pl.ds(wid * RPT + half * HR, HR)],
                              dst_v, sem_x)
        d1.wait()
        d2.wait()
        if half == 0:
            g_fire(0, buf_p, sem_gp)
            init_d.wait()
            plsc.subcore_barrier()
        else:
            g_fire(0, buf_p, sem_gp)
        lax.fori_loop(0, HR // 2, body, 0)
        s_drain(buf_q, sem_sq)

    plsc.subcore_barrier()
    pltpu.sync_copy(acc_sm.at[pl.ds(s * NPT, NPT)],
                    out_hbm.at[c, pl.ds(s * NPT, NPT)])


_scat_call = pl.kernel(
    _scat_body,
    out_type=jax.ShapeDtypeStruct((NC, NP, D), jnp.float32),
    mesh=_mesh,
    scratch_types=[
        pltpu.VMEM((HR, EC), jnp.int32),
        pltpu.VMEM((HR, EC), jnp.int32),
        pltpu.VMEM((EC, D), jnp.float32),
        pltpu.VMEM((EC, D), jnp.float32),
        pltpu.VMEM_SHARED((NP, D), jnp.float32),
        pltpu.SemaphoreType.DMA,
        pltpu.SemaphoreType.DMA,
        pltpu.SemaphoreType.DMA,
        pltpu.SemaphoreType.DMA,
        pltpu.SemaphoreType.DMA,
        pltpu.SemaphoreType.DMA,
    ],
)


def _mm_body(x_ref, w_ref, h_ref):
    h_ref[...] = jnp.dot(x_ref[...], w_ref[...],
                         preferred_element_type=jnp.float32)


_mm_call = pl.pallas_call(
    _mm_body,
    out_shape=jax.ShapeDtypeStruct((N, D), jnp.float32),
)


def _scale_body(h_ref, h0_ref, h1_ref, g_ref, dinv_ref):
    deg = 1.0 + h0_ref[...] + h1_ref[...]
    dinv = lax.rsqrt(deg)
    g_ref[pl.ds(0, N), :] = h_ref[...] * dinv[:N, :]
    g_ref[pl.ds(N, NP - N), :] = jnp.zeros((NP - N, D), jnp.float32)
    dinv_ref[...] = dinv


_scale_call = pl.pallas_call(
    _scale_body,
    out_shape=[
        jax.ShapeDtypeStruct((NP, D), jnp.float32),
        jax.ShapeDtypeStruct((NP, 1), jnp.float32),
    ],
)


def _fin_body(p0_ref, p1_ref, g_ref, dinv_ref, b_ref, o_ref):
    ssum = p0_ref[...] + p1_ref[...] - g_ref[...]
    o = ssum * dinv_ref[...] + b_ref[...]
    o_ref[...] = jnp.maximum(o, 0.0)


_fin_call = pl.pallas_call(
    _fin_body,
    out_shape=jax.ShapeDtypeStruct((N, D), jnp.float32),
)


def kernel(x, edge_index, W, b):
    src = edge_index[0]
    dst = edge_index[1]
    pad = EP - src.shape[0]
    pad_idx = (N + (jnp.arange(pad, dtype=jnp.int32) % (NP - N))).astype(jnp.int32)
    src_p = jnp.concatenate([src, pad_idx]).reshape(EROWS, EC)
    dst_p = jnp.concatenate([dst, pad_idx]).reshape(EROWS, EC)

    hist = _deg_call(dst_p)
    h = _mm_call(x, W)
    h0 = hist[0].reshape(NP, 1)
    h1 = hist[1].reshape(NP, 1)
    g, dinv = _scale_call(h, h0, h1)
    parts = _scat_call(g, src_p, dst_p)
    out = _fin_call(parts[0, :N], parts[1, :N], g[:N], dinv[:N],
                    b.reshape(1, D))
    return out


# speedup vs baseline: 44.2670x; 1.0497x over previous
import jax
import jax.numpy as jnp
from jax import lax
from jax.experimental import pallas as pl
from jax.experimental.pallas import tpu as pltpu
from jax.experimental.pallas import tpu_sc as plsc

N = 10000
D = 128
NP = 10240
NPT = NP // 16
EC = 128
RPT = 80
HR = RPT // 2
NTILES = 32
EROWS = NTILES * RPT
EP = EROWS * EC
NC, NS = 2, 16

_mesh = plsc.VectorSubcoreMesh(core_axis_name="c", subcore_axis_name="s")


def _deg_body(dst_hbm, out_hbm, idx_v, ones_v, zero_v, hist_sm, sem):
    c = lax.axis_index("c")
    s = lax.axis_index("s")
    wid = s * NC + c
    for i in range(EC // 16):
        ones_v[pl.ds(i * 16, 16)] = jnp.full((16,), 1.0, jnp.float32)
    for i in range(NPT // 16):
        zero_v[pl.ds(i * 16, 16)] = jnp.zeros((16,), jnp.float32)
    pltpu.sync_copy(zero_v, hist_sm.at[pl.ds(s * NPT, NPT)])
    plsc.subcore_barrier()
    pltpu.sync_copy(dst_hbm.at[pl.ds(wid * RPT, RPT)], idx_v)

    def body(j, carry):
        pltpu.async_copy(ones_v, hist_sm.at[idx_v.at[j]], sem, add=True)
        return carry

    lax.fori_loop(0, RPT, body, 0)
    pltpu.make_async_copy(dst_hbm.at[pl.ds(0, RPT)], idx_v, sem).wait()
    plsc.subcore_barrier()
    pltpu.sync_copy(hist_sm.at[pl.ds(s * NPT, NPT)],
                    out_hbm.at[c, pl.ds(s * NPT, NPT)])


_deg_call = pl.kernel(
    _deg_body,
    out_type=jax.ShapeDtypeStruct((NC, NP), jnp.float32),
    mesh=_mesh,
    scratch_types=[
        pltpu.VMEM((RPT, EC), jnp.int32),
        pltpu.VMEM((EC,), jnp.float32),
        pltpu.VMEM((NPT,), jnp.float32),
        pltpu.VMEM_SHARED((NP,), jnp.float32),
        pltpu.SemaphoreType.DMA,
    ],
)


def _scat_body(g_hbm, src_hbm, dst_hbm, out_hbm, src_v, dst_v, buf_p, buf_q,
               acc_sm, sem_i, sem_x, sem_gp, sem_gq, sem_sp, sem_sq):
    c = lax.axis_index("c")
    s = lax.axis_index("s")
    wid = s * NC + c
    init_d = pltpu.async_copy(g_hbm.at[pl.ds(s * NPT, NPT)],
                              acc_sm.at[pl.ds(s * NPT, NPT)], sem_i)

    def g_fire(j, buf, sem):
        pltpu.async_copy(g_hbm.at[src_v.at[j]], buf, sem)

    def g_drain(buf, sem):
        pltpu.make_async_copy(g_hbm.at[pl.ds(0, EC)], buf, sem).wait()

    def s_fire(j, buf, sem):
        pltpu.async_copy(buf, acc_sm.at[dst_v.at[j]], sem, add=True)

    def s_drain(buf, sem):
        pltpu.make_async_copy(g_hbm.at[pl.ds(0, EC)], buf, sem).wait()

    def body(i, carry):
        j0 = 2 * i
        j1 = 2 * i + 1

        @pl.when(i > 0)
        def _():
            s_drain(buf_q, sem_sq)
        g_fire(j1, buf_q, sem_gq)
        g_drain(buf_p, sem_gp)
        s_fire(j0, buf_p, sem_sp)
        g_drain(buf_q, sem_gq)
        s_drain(buf_p, sem_sp)

        @pl.when(j0 + 2 < HR)
        def _():
            g_fire(j0 + 2, buf_p, sem_gp)
        s_fire(j1, buf_q, sem_sq)
        return carry

    for half in range(2):
        d1 = pltpu.async_copy(src_hbm.at[pl.ds(wid * RPT + half * HR, HR)],
                              src_v, sem_x)
        d2 = pltpu.async_copy(dst_hbm.at[pl.ds(wid * RPT + half * HR, HR)],
                              dst_v, sem_x)
        d1.wait()
        d2.wait()
        if half == 0:
            g_fire(0, buf_p, sem_gp)
            init_d.wait()
            plsc.subcore_barrier()
        else:
            g_fire(0, buf_p, sem_gp)
        lax.fori_loop(0, HR // 2, body, 0)
        s_drain(buf_q, sem_sq)

    plsc.subcore_barrier()
    pltpu.sync_copy(acc_sm.at[pl.ds(s * NPT, NPT)],
                    out_hbm.at[c, pl.ds(s * NPT, NPT)])


_scat_call = pl.kernel(
    _scat_body,
    out_type=jax.ShapeDtypeStruct((NC, NP, D), jnp.float32),
    mesh=_mesh,
    scratch_types=[
        pltpu.VMEM((HR, EC), jnp.int32),
        pltpu.VMEM((HR, EC), jnp.int32),
        pltpu.VMEM((EC, D), jnp.float32),
        pltpu.VMEM((EC, D), jnp.float32),
        pltpu.VMEM_SHARED((NP, D), jnp.float32),
        pltpu.SemaphoreType.DMA,
        pltpu.SemaphoreType.DMA,
        pltpu.SemaphoreType.DMA,
        pltpu.SemaphoreType.DMA,
        pltpu.SemaphoreType.DMA,
        pltpu.SemaphoreType.DMA,
    ],
)


def _mm_body(x_ref, w_ref, h_ref):
    h_ref[...] = jnp.dot(x_ref[...], w_ref[...],
                         preferred_element_type=jnp.float32)


_mm_call = pl.pallas_call(
    _mm_body,
    out_shape=jax.ShapeDtypeStruct((N, D), jnp.float32),
)


def _scale_body(h_ref, h0_ref, h1_ref, g_ref, dinv_ref):
    deg = 1.0 + h0_ref[...] + h1_ref[...]
    dinv = lax.rsqrt(deg)
    g_ref[pl.ds(0, N), :] = h_ref[...] * dinv[:N, :]
    g_ref[pl.ds(N, NP - N), :] = jnp.zeros((NP - N, D), jnp.float32)
    dinv_ref[...] = dinv


_scale_call = pl.pallas_call(
    _scale_body,
    out_shape=[
        jax.ShapeDtypeStruct((NP, D), jnp.float32),
        jax.ShapeDtypeStruct((NP, 1), jnp.float32),
    ],
)


def _fin_body(p_ref, g_ref, dinv_ref, b_ref, o_ref):
    ssum = p_ref[0, :N, :] + p_ref[1, :N, :] - g_ref[:N, :]
    o = ssum * dinv_ref[:N, :] + b_ref[...]
    o_ref[...] = jnp.maximum(o, 0.0)


_fin_call = pl.pallas_call(
    _fin_body,
    out_shape=jax.ShapeDtypeStruct((N, D), jnp.float32),
)


def kernel(x, edge_index, W, b):
    src = edge_index[0]
    dst = edge_index[1]
    pad = EP - src.shape[0]
    pad_idx = (N + (jnp.arange(pad, dtype=jnp.int32) % (NP - N))).astype(jnp.int32)
    src_p = jnp.concatenate([src, pad_idx]).reshape(EROWS, EC)
    dst_p = jnp.concatenate([dst, pad_idx]).reshape(EROWS, EC)

    hist = _deg_call(dst_p)
    h = _mm_call(x, W)
    h0 = hist[0].reshape(NP, 1)
    h1 = hist[1].reshape(NP, 1)
    g, dinv = _scale_call(h, h0, h1)
    parts = _scat_call(g, src_p, dst_p)
    out = _fin_call(parts, g, dinv, b.reshape(1, D))
    return out


# speedup vs baseline: 44.3937x; 1.0029x over previous
import jax
import jax.numpy as jnp
from jax import lax
from jax.experimental import pallas as pl
from jax.experimental.pallas import tpu as pltpu
from jax.experimental.pallas import tpu_sc as plsc

N = 10000
D = 128
NP = 10240
NPT = NP // 16
EC = 128
RPT = 80
HR = RPT // 2
NTILES = 32
EROWS = NTILES * RPT
EP = EROWS * EC
NC, NS = 2, 16

_mesh = plsc.VectorSubcoreMesh(core_axis_name="c", subcore_axis_name="s")


def _deg_body(dst_hbm, out_hbm, idx_v, ones_v, zero_v, hist_sm, sem):
    c = lax.axis_index("c")
    s = lax.axis_index("s")
    wid = s * NC + c
    for i in range(EC // 16):
        ones_v[pl.ds(i * 16, 16)] = jnp.full((16,), 1.0, jnp.float32)
    for i in range(NPT // 16):
        zero_v[pl.ds(i * 16, 16)] = jnp.zeros((16,), jnp.float32)
    pltpu.sync_copy(zero_v, hist_sm.at[pl.ds(s * NPT, NPT)])
    plsc.subcore_barrier()
    pltpu.sync_copy(dst_hbm.at[pl.ds(wid * RPT, RPT)], idx_v)

    def body(j, carry):
        pltpu.async_copy(ones_v, hist_sm.at[idx_v.at[j]], sem, add=True)
        return carry

    lax.fori_loop(0, RPT, body, 0)
    pltpu.make_async_copy(dst_hbm.at[pl.ds(0, RPT)], idx_v, sem).wait()
    plsc.subcore_barrier()
    pltpu.sync_copy(hist_sm.at[pl.ds(s * NPT, NPT)],
                    out_hbm.at[c, pl.ds(s * NPT, NPT)])


_deg_call = pl.kernel(
    _deg_body,
    out_type=jax.ShapeDtypeStruct((NC, NP), jnp.float32),
    mesh=_mesh,
    scratch_types=[
        pltpu.VMEM((RPT, EC), jnp.int32),
        pltpu.VMEM((EC,), jnp.float32),
        pltpu.VMEM((NPT,), jnp.float32),
        pltpu.VMEM_SHARED((NP,), jnp.float32),
        pltpu.SemaphoreType.DMA,
    ],
)


def _scat_body(g_hbm, src_hbm, dst_hbm, out_hbm, src_v, dst_v, buf_p, buf_q,
               acc_sm, sem_i, sem_x, sem_gp, sem_gq, sem_sp, sem_sq):
    c = lax.axis_index("c")
    s = lax.axis_index("s")
    wid = s * NC + c
    init_d = pltpu.async_copy(g_hbm.at[pl.ds(s * NPT, NPT)],
                              acc_sm.at[pl.ds(s * NPT, NPT)], sem_i)

    def g_fire(j, buf, sem):
        pltpu.async_copy(g_hbm.at[src_v.at[j]], buf, sem)

    def g_drain(buf, sem):
        pltpu.make_async_copy(g_hbm.at[pl.ds(0, EC)], buf, sem).wait()

    def s_fire(j, buf, sem):
        pltpu.async_copy(buf, acc_sm.at[dst_v.at[j]], sem, add=True)

    def s_drain(buf, sem):
        pltpu.make_async_copy(g_hbm.at[pl.ds(0, EC)], buf, sem).wait()

    def body(i, carry):
        j0 = 2 * i
        j1 = 2 * i + 1

        @pl.when(i > 0)
        def _():
            s_drain(buf_q, sem_sq)
        g_fire(j1, buf_q, sem_gq)
        g_drain(buf_p, sem_gp)
        s_fire(j0, buf_p, sem_sp)
        g_drain(buf_q, sem_gq)
        s_drain(buf_p, sem_sp)

        @pl.when(j0 + 2 < HR)
        def _():
            g_fire(j0 + 2, buf_p, sem_gp)
        s_fire(j1, buf_q, sem_sq)
        return carry

    for half in range(2):
        d1 = pltpu.async_copy(src_hbm.at[pl.ds(wid * RPT + half * HR, HR)],
                              src_v, sem_x)
        d2 = pltpu.async_copy(dst_hbm.at[pl.ds(wid * RPT + half * HR, HR)],
                              dst_v, sem_x)
        d1.wait()
        d2.wait()
        if half == 0:
            g_fire(0, buf_p, sem_gp)
            init_d.wait()
            plsc.subcore_barrier()
        else:
            g_fire(0, buf_p, sem_gp)
        lax.fori_loop(0, HR // 2, body, 0)
        s_drain(buf_q, sem_sq)

    plsc.subcore_barrier()
    pltpu.sync_copy(acc_sm.at[pl.ds(s * NPT, NPT)],
                    out_hbm.at[c, pl.ds(s * NPT, NPT)])


_scat_call = pl.kernel(
    _scat_body,
    out_type=jax.ShapeDtypeStruct((NC, NP, D), jnp.float32),
    mesh=_mesh,
    scratch_types=[
        pltpu.VMEM((HR, EC), jnp.int32),
        pltpu.VMEM((HR, EC), jnp.int32),
        pltpu.VMEM((EC, D), jnp.float32),
        pltpu.VMEM((EC, D), jnp.float32),
        pltpu.VMEM_SHARED((NP, D), jnp.float32),
        pltpu.SemaphoreType.DMA,
        pltpu.SemaphoreType.DMA,
        pltpu.SemaphoreType.DMA,
        pltpu.SemaphoreType.DMA,
        pltpu.SemaphoreType.DMA,
        pltpu.SemaphoreType.DMA,
    ],
)


def _mm_body(x_ref, w_ref, h_ref):
    h_ref[...] = jnp.dot(x_ref[...], w_ref[...],
                         preferred_element_type=jnp.float32)


_mm_call = pl.pallas_call(
    _mm_body,
    out_shape=jax.ShapeDtypeStruct((N, D), jnp.float32),
)


def _scale_body(h_ref, h0_ref, h1_ref, g_ref, dinv_ref):
    deg = 1.0 + h0_ref[...] + h1_ref[...]
    dinv = lax.rsqrt(deg)
    g_ref[pl.ds(0, N), :] = h_ref[...] * dinv[:N, :]
    g_ref[pl.ds(N, NP - N), :] = jnp.zeros((NP - N, D), jnp.float32)
    dinv_ref[...] = dinv


_scale_call = pl.pallas_call(
    _scale_body,
    out_shape=[
        jax.ShapeDtypeStruct((NP, D), jnp.float32),
        jax.ShapeDtypeStruct((NP, 1), jnp.float32),
    ],
)


def _fin_body(p_ref, g_ref, dinv_ref, b_ref, o_ref):
    ssum = p_ref[0, :N, :] + p_ref[1, :N, :] - g_ref[:N, :]
    o = ssum * dinv_ref[:N, :] + b_ref[...]
    o_ref[...] = jnp.maximum(o, 0.0)


_fin_call = pl.pallas_call(
    _fin_body,
    out_shape=jax.ShapeDtypeStruct((N, D), jnp.float32),
)


def kernel(x, edge_index, W, b):
    src = edge_index[0]
    dst = edge_index[1]
    pad = EP - src.shape[0]
    pad_idx = (N + (jnp.arange(pad, dtype=jnp.int32) % (NP - N))).astype(jnp.int32)
    src_p = jnp.concatenate([src, pad_idx]).reshape(EROWS, EC)
    dst_p = jnp.concatenate([dst, pad_idx]).reshape(EROWS, EC)

    h = _mm_call(x, W)
    hist = _deg_call(dst_p)
    h0 = hist[0].reshape(NP, 1)
    h1 = hist[1].reshape(NP, 1)
    g, dinv = _scale_call(h, h0, h1)
    parts = _scat_call(g, src_p, dst_p)
    out = _fin_call(parts, g, dinv, b.reshape(1, D))
    return out
